# Initial kernel scaffold; baseline (speedup 1.0000x reference)
import jax
import jax.numpy as jnp
from jax.experimental import pallas as pl


def kernel(loc_pred, cls_pred, gt_boxes, gt_labels, default_boxes):
    raise NotImplementedError("write your pallas kernel here")


# speedup vs baseline: 43.0119x; 43.0119x over previous
import jax
import jax.numpy as jnp
from jax.experimental import pallas as pl

B = 64
D = 8732
C = 81
O = 16
THR = 0.5
NEG_POS = 3
ALPHA = 1.0


def _match_kernel(gx1_ref, gy1_ref, gx2_ref, gy2_ref, glab_ref,
                  dcx_ref, dcy_ref, dw_ref, dh_ref,
                  lp0_ref, lp1_ref, lp2_ref, lp3_ref,
                  labels_ref, sl1_ref):
    dcx = dcx_ref[...]
    dcy = dcy_ref[...]
    dw = dw_ref[...]
    dh = dh_ref[...]
    dx1 = dcx - dw / 2.0
    dy1 = dcy - dh / 2.0
    dx2 = dcx + dw / 2.0
    dy2 = dcy + dh / 2.0
    area_d = (dx2 - dx1) * (dy2 - dy1)

    nrows = gx1_ref.shape[0]
    lane = jax.lax.broadcasted_iota(jnp.int32, (nrows, D), 1)

    best = jnp.full((nrows, D), -1.0, jnp.float32)
    opd = jnp.zeros((nrows, D), jnp.int32)
    dpg = []
    for j in range(O):
        gx1 = gx1_ref[:, j:j + 1]
        gy1 = gy1_ref[:, j:j + 1]
        gx2 = gx2_ref[:, j:j + 1]
        gy2 = gy2_ref[:, j:j + 1]
        ltx = jnp.maximum(gx1, dx1)
        lty = jnp.maximum(gy1, dy1)
        rbx = jnp.minimum(gx2, dx2)
        rby = jnp.minimum(gy2, dy2)
        inter = jnp.maximum(rbx - ltx, 0.0) * jnp.maximum(rby - lty, 0.0)
        area_g = (gx2 - gx1) * (gy2 - gy1)
        union = area_g + area_d - inter
        iou = inter / jnp.maximum(union, 1e-10)
        upd = iou > best
        best = jnp.where(upd, iou, best)
        opd = jnp.where(upd, j, opd)
        m = jnp.max(iou, axis=1, keepdims=True)
        dpg.append(jnp.min(jnp.where(iou == m, lane, D), axis=1, keepdims=True))

    for j in range(O):
        force = lane == dpg[j]
        opd = jnp.where(force, j, opd)
        best = jnp.where(force, 1.0, best)

    lab = jnp.zeros((nrows, D), jnp.int32)
    mx1 = jnp.zeros((nrows, D), jnp.float32)
    my1 = jnp.zeros((nrows, D), jnp.float32)
    mx2 = jnp.zeros((nrows, D), jnp.float32)
    my2 = jnp.zeros((nrows, D), jnp.float32)
    for j in range(O):
        sel = opd == j
        lab = jnp.where(sel, glab_ref[:, j:j + 1], lab)
        mx1 = jnp.where(sel, gx1_ref[:, j:j + 1], mx1)
        my1 = jnp.where(sel, gy1_ref[:, j:j + 1], my1)
        mx2 = jnp.where(sel, gx2_ref[:, j:j + 1], mx2)
        my2 = jnp.where(sel, gy2_ref[:, j:j + 1], my2)
    lab = jnp.where(best < THR, 0, lab)
    posf = (lab > 0).astype(jnp.float32)

    cx = (mx1 + mx2) / 2.0
    cy = (my1 + my2) / 2.0
    w = mx2 - mx1
    h = my2 - my1
    ecx = (cx - dcx) / (dw / 10.0)
    ecy = (cy - dcy) / (dh / 10.0)
    ew = jnp.log(jnp.maximum(w, 1e-6) / dw) * 5.0
    eh = jnp.log(jnp.maximum(h, 1e-6) / dh) * 5.0

    s = jnp.zeros((nrows, 1), jnp.float32)
    for lp_ref, e in ((lp0_ref, ecx), (lp1_ref, ecy), (lp2_ref, ew), (lp3_ref, eh)):
        diff = lp_ref[...] - e
        ad = jnp.abs(diff)
        sl1 = jnp.where(ad < 1.0, 0.5 * diff * diff, ad - 0.5)
        s = s + jnp.sum(sl1 * posf, axis=1, keepdims=True)

    labels_ref[...] = lab
    sl1_ref[...] = s


def _ce_kernel(cls_ref, lab_ref, ce_ref):
    x = cls_ref[0]
    lab = lab_ref[0]
    sumexp = jnp.sum(jnp.exp(x), axis=0, keepdims=True)
    cls_iota = jax.lax.broadcasted_iota(jnp.int32, (C, D), 0)
    xsel = jnp.sum(jnp.where(cls_iota == lab, x, 0.0), axis=0, keepdims=True)
    ce_ref[0] = jnp.log(sumexp) - xsel


def _loss_kernel(ce_ref, lab_ref, sl1_ref, out_ref):
    ce = ce_ref[...]
    lab = lab_ref[...]
    pos = lab > 0
    posf = pos.astype(jnp.float32)
    n_pos = jnp.sum(posf, axis=1, keepdims=True)
    conf_pos = jnp.sum(ce * posf, axis=(0, 1), keepdims=True)
    ce_neg = jnp.where(pos, 0.0, ce)
    v = jax.lax.bitcast_convert_type(ce_neg, jnp.int32)
    ki = jnp.minimum(n_pos.astype(jnp.int32) * NEG_POS, D)
    prefix = jnp.zeros((B, 1), jnp.int32)
    for bit in range(30, -1, -1):
        cand = prefix | (1 << bit)
        cnt = jnp.sum((v >= cand).astype(jnp.int32), axis=1, keepdims=True)
        prefix = jnp.where(cnt >= ki, cand, prefix)
    gt_mask = v > prefix
    cnt_gt = jnp.sum(gt_mask.astype(jnp.float32), axis=1, keepdims=True)
    sum_gt = jnp.sum(jnp.where(gt_mask, ce_neg, 0.0), axis=1, keepdims=True)
    tf = jax.lax.bitcast_convert_type(prefix, jnp.float32)
    conf_hard = jnp.sum(sum_gt + (ki.astype(jnp.float32) - cnt_gt) * tf,
                        axis=(0, 1), keepdims=True)
    total_pos = jnp.maximum(jnp.sum(n_pos, axis=(0, 1), keepdims=True), 1.0)
    sl1_total = jnp.sum(sl1_ref[...], axis=(0, 1), keepdims=True)
    out_ref[...] = (conf_pos + conf_hard) / total_pos \
        + ALPHA * sl1_total / (total_pos * 4.0)


def kernel(loc_pred, cls_pred, gt_boxes, gt_labels, default_boxes):
    gx1 = gt_boxes[:, :, 0]
    gy1 = gt_boxes[:, :, 1]
    gx2 = gt_boxes[:, :, 2]
    gy2 = gt_boxes[:, :, 3]
    glab = gt_labels.astype(jnp.int32)
    dcx = default_boxes[:, 0].reshape(1, D)
    dcy = default_boxes[:, 1].reshape(1, D)
    dw = default_boxes[:, 2].reshape(1, D)
    dh = default_boxes[:, 3].reshape(1, D)
    lp0 = loc_pred[:, :, 0]
    lp1 = loc_pred[:, :, 1]
    lp2 = loc_pred[:, :, 2]
    lp3 = loc_pred[:, :, 3]

    rows = 8
    g_spec = pl.BlockSpec((rows, O), lambda i: (i, 0))
    d_spec = pl.BlockSpec((1, D), lambda i: (0, 0))
    lp_spec = pl.BlockSpec((rows, D), lambda i: (i, 0))
    labels, sl1 = pl.pallas_call(
        _match_kernel,
        grid=(B // rows,),
        in_specs=[g_spec, g_spec, g_spec, g_spec, g_spec,
                  d_spec, d_spec, d_spec, d_spec,
                  lp_spec, lp_spec, lp_spec, lp_spec],
        out_specs=[pl.BlockSpec((rows, D), lambda i: (i, 0)),
                   pl.BlockSpec((rows, 1), lambda i: (i, 0))],
        out_shape=[jax.ShapeDtypeStruct((B, D), jnp.int32),
                   jax.ShapeDtypeStruct((B, 1), jnp.float32)],
    )(gx1, gy1, gx2, gy2, glab, dcx, dcy, dw, dh, lp0, lp1, lp2, lp3)

    cls_t = jnp.transpose(cls_pred, (0, 2, 1))
    ce3 = pl.pallas_call(
        _ce_kernel,
        grid=(B,),
        in_specs=[pl.BlockSpec((1, C, D), lambda i: (i, 0, 0)),
                  pl.BlockSpec((1, 1, D), lambda i: (i, 0, 0))],
        out_specs=pl.BlockSpec((1, 1, D), lambda i: (i, 0, 0)),
        out_shape=jax.ShapeDtypeStruct((B, 1, D), jnp.float32),
    )(cls_t, labels.reshape(B, 1, D))

    loss = pl.pallas_call(
        _loss_kernel,
        out_shape=jax.ShapeDtypeStruct((1, 1), jnp.float32),
    )(ce3.reshape(B, D), labels, sl1)
    return loss.reshape(())
